# baseline (speedup 1.0000x reference)
import dataclasses
import functools

import jax
import jax.numpy as jnp
from jax import lax
from jax.experimental import pallas as pl
from jax.experimental.pallas import tpu as pltpu
from jax.experimental.pallas import tpu_sc as plsc

_PAD = 1
_EPS = 1e-05
_HIDDEN = 768

_NC = 2
_NS = 16
_NW = _NC * _NS
_CH = 32


def _posid_body(ids_ref, oid_ref):
    ids = ids_ref[...]
    mask = (ids != _PAD).astype(jnp.int32)
    x = mask
    seq = ids.shape[1]
    k = 1
    while k < seq:
        shifted = jnp.concatenate(
            [jnp.zeros((ids.shape[0], k), jnp.int32), x[:, :-k]], axis=1)
        x = x + shifted
        k *= 2
    oid_ref[...] = x * mask + _PAD


def _position_ids(input_ids):
    return pl.pallas_call(
        _posid_body,
        out_shape=jax.ShapeDtypeStruct(input_ids.shape, jnp.int32),
    )(input_ids)


def _pack_body(pos_ref, out_ref):
    p = pos_ref[...].astype(jnp.bfloat16)
    v, dcb = p.shape
    p4 = p.reshape(v, dcb // 32, 2, 16)
    lo = lax.bitcast_convert_type(p4[:, :, 0, :], jnp.int16)
    hi = lax.bitcast_convert_type(p4[:, :, 1, :], jnp.int16)
    lo32 = lo.astype(jnp.int32) & 0xFFFF
    hi32 = hi.astype(jnp.int32) << 16
    out_ref[...] = (lo32 | hi32).reshape(v, dcb // 2)


def _pack_pos(pos_emb):
    v, d = pos_emb.shape
    cb = 256
    return pl.pallas_call(
        _pack_body,
        grid=(d // cb,),
        in_specs=[pl.BlockSpec((v, cb), lambda i: (0, i))],
        out_specs=pl.BlockSpec((v, cb // 2), lambda i: (0, i)),
        out_shape=jax.ShapeDtypeStruct((v, d // 2), jnp.int32),
    )(pos_emb)


@functools.lru_cache(maxsize=None)
def _make_gather_add(v_word, v_pos, d, b):
    rpw = b // _NW
    nch = rpw // _CH
    assert nch % 2 == 0
    mesh = plsc.VectorSubcoreMesh(core_axis_name="c", subcore_axis_name="s")
    cp = pltpu.CompilerParams()
    if "needs_layout_passes" in pltpu.CompilerParams.__dataclass_fields__:
        cp = dataclasses.replace(cp, needs_layout_passes=False)

    @functools.partial(
        pl.kernel,
        mesh=mesh,
        compiler_params=cp,
        out_type=jax.ShapeDtypeStruct((b, d), jnp.float32),
        scratch_types=[
            pltpu.VMEM((rpw,), jnp.int32),
            pltpu.VMEM((rpw,), jnp.int32),
            pltpu.VMEM((2, _CH, d), jnp.float32),
            pltpu.VMEM((2, _CH, d // 2), jnp.int32),
        ] + [pltpu.SemaphoreType.DMA] * 6,
    )
    def gather_kernel(word_hbm, pos_hbm, iw_hbm, ip_hbm, out_hbm,
                      iw_v, ip_v, wbuf, pbuf,
                      sw0, sw1, sp0, sp1, so0, so1):
        semw = (sw0, sw1)
        semp = (sp0, sp1)
        semo = (so0, so1)
        wid = lax.axis_index("s") * _NC + lax.axis_index("c")
        base = wid * rpw

        def fire(cc, bb):
            pltpu.async_copy(
                word_hbm.at[iw_v.at[pl.ds(cc * _CH, _CH)]], wbuf.at[bb],
                semw[bb])
            pltpu.async_copy(
                pos_hbm.at[ip_v.at[pl.ds(cc * _CH, _CH)]], pbuf.at[bb],
                semp[bb])

        def wait_gather(bb):
            pltpu.make_async_copy(
                word_hbm.at[pl.ds(0, _CH)], wbuf.at[bb], semw[bb]).wait()
            pltpu.make_async_copy(
                pos_hbm.at[pl.ds(0, _CH)], pbuf.at[bb], semp[bb]).wait()

        def wait_out(bb):
            pltpu.make_async_copy(
                wbuf.at[bb], out_hbm.at[pl.ds(base, _CH)], semo[bb]).wait()

        pltpu.sync_copy(iw_hbm.at[pl.ds(base, rpw)], iw_v)
        pltpu.sync_copy(ip_hbm.at[pl.ds(base, rpw)], ip_v)
        fire(0, 0)

        @pl.loop(0, nch, step=2)
        def _(c):
            for bb in range(2):
                cc = c + bb
                nb = 1 - bb

                @pl.when(cc + 1 < nch)
                def _():
                    @pl.when(cc >= 1)
                    def _():
                        wait_out(nb)

                    fire(cc + 1, nb)

                wait_gather(bb)

                @plsc.parallel_loop(0, _CH, step=1, unroll=2)
                def _(r):
                    for col in range(0, d, 32):
                        x32 = pbuf[bb, r, pl.ds(col // 2, 16)]
                        lo = plsc.bitcast(x32 << 16, jnp.float32)
                        hi = plsc.bitcast(x32 & jnp.int32(-65536),
                                          jnp.float32)
                        wbuf[bb, r, pl.ds(col, 16)] = (
                            wbuf[bb, r, pl.ds(col, 16)] + lo)
                        wbuf[bb, r, pl.ds(col + 16, 16)] = (
                            wbuf[bb, r, pl.ds(col + 16, 16)] + hi)

                pltpu.async_copy(
                    wbuf.at[bb], out_hbm.at[pl.ds(base + cc * _CH, _CH)],
                    semo[bb])

        wait_out(0)
        wait_out(1)

    return gather_kernel


def _ln_body(s_ref, t_ref, lw_ref, lb_ref, o_ref):
    x = s_ref[...] + t_ref[0:1, :]
    mean = jnp.mean(x, axis=-1, keepdims=True)
    m2 = jnp.mean(x * x, axis=-1, keepdims=True)
    var = m2 - mean * mean
    o_ref[...] = (x - mean) * lax.rsqrt(var + _EPS) * lw_ref[...] + lb_ref[...]


def _ln(sum_rows, type_emb, ln_w, ln_b):
    b, d = sum_rows.shape
    rb = 2048
    grid = (b // rb,)
    return pl.pallas_call(
        _ln_body,
        grid=grid,
        in_specs=[
            pl.BlockSpec((rb, d), lambda i: (i, 0)),
            pl.BlockSpec(type_emb.shape, lambda i: (0, 0)),
            pl.BlockSpec((1, d), lambda i: (0, 0)),
            pl.BlockSpec((1, d), lambda i: (0, 0)),
        ],
        out_specs=pl.BlockSpec((rb, d), lambda i: (i, 0)),
        out_shape=jax.ShapeDtypeStruct((b, d), jnp.float32),
    )(sum_rows, type_emb, ln_w, ln_b)


def kernel(input_ids, word_emb, pos_emb, type_emb, ln_w, ln_b):
    bb, seq = input_ids.shape
    d = word_emb.shape[1]
    b = bb * seq

    position_ids = _position_ids(input_ids)
    pos_packed = _pack_pos(pos_emb)
    ids_flat = input_ids.reshape(b)
    pos_flat = position_ids.reshape(b)

    gather = _make_gather_add(word_emb.shape[0], pos_emb.shape[0], d, b)
    sum_rows = gather(word_emb, pos_packed, ids_flat, pos_flat)

    out = _ln(sum_rows, type_emb,
              ln_w.reshape(1, d), ln_b.reshape(1, d))
    return out.reshape(bb, seq, d)


# speedup vs baseline: 2.0484x; 2.0484x over previous
import dataclasses
import functools

import jax
import jax.numpy as jnp
from jax import lax
from jax.experimental import pallas as pl
from jax.experimental.pallas import tpu as pltpu
from jax.experimental.pallas import tpu_sc as plsc

_PAD = 1
_EPS = 1e-05
_HIDDEN = 768

_NC = 2
_NS = 16
_NW = _NC * _NS
_CH = 16
_NB = 4


def _posid_body(ids_ref, oid_ref):
    ids = ids_ref[...]
    mask = (ids != _PAD).astype(jnp.int32)
    x = mask
    seq = ids.shape[1]
    k = 1
    while k < seq:
        shifted = jnp.concatenate(
            [jnp.zeros((ids.shape[0], k), jnp.int32), x[:, :-k]], axis=1)
        x = x + shifted
        k *= 2
    oid_ref[...] = x * mask + _PAD


def _position_ids(input_ids):
    return pl.pallas_call(
        _posid_body,
        out_shape=jax.ShapeDtypeStruct(input_ids.shape, jnp.int32),
    )(input_ids)


@functools.lru_cache(maxsize=None)
def _make_gather_add(v_word, v_pos, d, b):
    rpw = b // _NW
    nch = rpw // _CH
    assert nch % _NB == 0 and nch >= 2 * _NB
    mesh = plsc.VectorSubcoreMesh(core_axis_name="c", subcore_axis_name="s")
    cp = pltpu.CompilerParams()
    if "needs_layout_passes" in pltpu.CompilerParams.__dataclass_fields__:
        cp = dataclasses.replace(cp, needs_layout_passes=False)

    @functools.partial(
        pl.kernel,
        mesh=mesh,
        compiler_params=cp,
        out_type=jax.ShapeDtypeStruct((b, d), jnp.float32),
        scratch_types=[
            pltpu.VMEM((rpw,), jnp.int32),
            pltpu.VMEM((rpw,), jnp.int32),
            pltpu.VMEM((_NB, _CH, d), jnp.float32),
            pltpu.VMEM((_NB, _CH, d), jnp.float32),
        ] + [pltpu.SemaphoreType.DMA] * (3 * _NB),
    )
    def gather_kernel(word_hbm, pos_hbm, iw_hbm, ip_hbm, out_hbm,
                      iw_v, ip_v, wbuf, pbuf, *sems):
        semw = sems[0:_NB]
        semp = sems[_NB:2 * _NB]
        semo = sems[2 * _NB:3 * _NB]
        wid = lax.axis_index("s") * _NC + lax.axis_index("c")
        base = wid * rpw

        def fire(cc, bb):
            pltpu.async_copy(
                word_hbm.at[iw_v.at[pl.ds(cc * _CH, _CH)]], wbuf.at[bb],
                semw[bb])
            pltpu.async_copy(
                pos_hbm.at[ip_v.at[pl.ds(cc * _CH, _CH)]], pbuf.at[bb],
                semp[bb])

        def wait_gather(bb):
            pltpu.make_async_copy(
                word_hbm.at[pl.ds(0, _CH)], wbuf.at[bb], semw[bb]).wait()
            pltpu.make_async_copy(
                pos_hbm.at[pl.ds(0, _CH)], pbuf.at[bb], semp[bb]).wait()

        def wait_out(bb):
            pltpu.make_async_copy(
                wbuf.at[bb], out_hbm.at[pl.ds(base, _CH)], semo[bb]).wait()

        pltpu.sync_copy(iw_hbm.at[pl.ds(base, rpw)], iw_v)
        pltpu.sync_copy(ip_hbm.at[pl.ds(base, rpw)], ip_v)
        fire(0, 0)
        fire(1, 1)

        @pl.loop(0, nch, step=_NB)
        def _(c):
            for bb in range(_NB):
                cc = c + bb
                fb = (bb + 2) % _NB

                @pl.when(cc + 2 < nch)
                def _():
                    @pl.when(cc >= 2)
                    def _():
                        wait_out(fb)

                    fire(cc + 2, fb)

                wait_gather(bb)

                @plsc.parallel_loop(0, _CH, step=1, unroll=2)
                def _(r):
                    for col in range(0, d, 16):
                        wbuf[bb, r, pl.ds(col, 16)] = (
                            wbuf[bb, r, pl.ds(col, 16)]
                            + pbuf[bb, r, pl.ds(col, 16)])

                pltpu.async_copy(
                    wbuf.at[bb], out_hbm.at[pl.ds(base + cc * _CH, _CH)],
                    semo[bb])

        for bb in range(_NB):
            wait_out(bb)

    return gather_kernel


def _ln_body(s_ref, t_ref, lw_ref, lb_ref, o_ref):
    x = s_ref[...] + t_ref[0:1, :]
    mean = jnp.mean(x, axis=-1, keepdims=True)
    m2 = jnp.mean(x * x, axis=-1, keepdims=True)
    var = m2 - mean * mean
    o_ref[...] = (x - mean) * lax.rsqrt(var + _EPS) * lw_ref[...] + lb_ref[...]


def _ln(sum_rows, type_emb, ln_w, ln_b):
    b, d = sum_rows.shape
    rb = 2048
    grid = (b // rb,)
    return pl.pallas_call(
        _ln_body,
        grid=grid,
        in_specs=[
            pl.BlockSpec((rb, d), lambda i: (i, 0)),
            pl.BlockSpec(type_emb.shape, lambda i: (0, 0)),
            pl.BlockSpec((1, d), lambda i: (0, 0)),
            pl.BlockSpec((1, d), lambda i: (0, 0)),
        ],
        out_specs=pl.BlockSpec((rb, d), lambda i: (i, 0)),
        out_shape=jax.ShapeDtypeStruct((b, d), jnp.float32),
    )(sum_rows, type_emb, ln_w, ln_b)


def kernel(input_ids, word_emb, pos_emb, type_emb, ln_w, ln_b):
    bb, seq = input_ids.shape
    d = word_emb.shape[1]
    b = bb * seq

    position_ids = _position_ids(input_ids)
    ids_flat = input_ids.reshape(b)
    pos_flat = position_ids.reshape(b)

    gather = _make_gather_add(word_emb.shape[0], pos_emb.shape[0], d, b)
    sum_rows = gather(word_emb, pos_emb, ids_flat, pos_flat)

    out = _ln(sum_rows, type_emb,
              ln_w.reshape(1, d), ln_b.reshape(1, d))
    return out.reshape(bb, seq, d)


# speedup vs baseline: 2.0534x; 1.0025x over previous
import dataclasses
import functools

import jax
import jax.numpy as jnp
from jax import lax
from jax.experimental import pallas as pl
from jax.experimental.pallas import tpu as pltpu
from jax.experimental.pallas import tpu_sc as plsc

_PAD = 1
_EPS = 1e-05
_HIDDEN = 768

_NC = 2
_NS = 16
_NW = _NC * _NS
_CH = 16
_NB = 4


def _posid_body(ids_ref, oid_ref):
    ids = ids_ref[...]
    mask = (ids != _PAD).astype(jnp.int32)
    x = mask
    seq = ids.shape[1]
    k = 1
    while k < seq:
        shifted = jnp.concatenate(
            [jnp.zeros((ids.shape[0], k), jnp.int32), x[:, :-k]], axis=1)
        x = x + shifted
        k *= 2
    oid_ref[...] = x * mask + _PAD


def _position_ids(input_ids):
    return pl.pallas_call(
        _posid_body,
        out_shape=jax.ShapeDtypeStruct(input_ids.shape, jnp.int32),
    )(input_ids)


@functools.lru_cache(maxsize=None)
def _make_gather_add(v_word, v_pos, d, nrow, seq):
    b = nrow * seq
    rpw = b // _NW
    nch = rpw // _CH
    wps = seq // rpw
    assert nch % _NB == 0 and nch >= 2 * _NB and wps * rpw == seq
    mesh = plsc.VectorSubcoreMesh(core_axis_name="c", subcore_axis_name="s")
    cp = pltpu.CompilerParams()
    if "needs_layout_passes" in pltpu.CompilerParams.__dataclass_fields__:
        cp = dataclasses.replace(cp, needs_layout_passes=False)

    @functools.partial(
        pl.kernel,
        mesh=mesh,
        compiler_params=cp,
        out_type=jax.ShapeDtypeStruct((b, d), jnp.float32),
        scratch_types=[
            pltpu.VMEM((rpw,), jnp.int32),
            pltpu.VMEM((rpw,), jnp.int32),
            pltpu.VMEM((_NB, _CH, d), jnp.float32),
            pltpu.VMEM((_NB, _CH, d), jnp.float32),
        ] + [pltpu.SemaphoreType.DMA] * (3 * _NB),
    )
    def gather_kernel(word_hbm, pos_hbm, iw_hbm, ip_hbm, out_hbm,
                      iw_v, ip_v, wbuf, pbuf, *sems):
        semw = sems[0:_NB]
        semp = sems[_NB:2 * _NB]
        semo = sems[2 * _NB:3 * _NB]
        wid = lax.axis_index("s") * _NC + lax.axis_index("c")
        base = wid * rpw
        srow = wid // wps
        scol = (wid % wps) * rpw

        def fire(cc, bb):
            pltpu.async_copy(
                word_hbm.at[iw_v.at[pl.ds(cc * _CH, _CH)]], wbuf.at[bb],
                semw[bb])
            pltpu.async_copy(
                pos_hbm.at[ip_v.at[pl.ds(cc * _CH, _CH)]], pbuf.at[bb],
                semp[bb])

        def wait_gather(bb):
            pltpu.make_async_copy(
                word_hbm.at[pl.ds(0, _CH)], wbuf.at[bb], semw[bb]).wait()
            pltpu.make_async_copy(
                pos_hbm.at[pl.ds(0, _CH)], pbuf.at[bb], semp[bb]).wait()

        def wait_out(bb):
            pltpu.make_async_copy(
                wbuf.at[bb], out_hbm.at[pl.ds(base, _CH)], semo[bb]).wait()

        pltpu.sync_copy(iw_hbm.at[srow, pl.ds(scol, rpw)], iw_v)
        pltpu.sync_copy(ip_hbm.at[srow, pl.ds(scol, rpw)], ip_v)
        fire(0, 0)
        fire(1, 1)

        @pl.loop(0, nch, step=_NB)
        def _(c):
            for bb in range(_NB):
                cc = c + bb
                fb = (bb + 2) % _NB

                @pl.when(cc + 2 < nch)
                def _():
                    @pl.when(cc >= 2)
                    def _():
                        wait_out(fb)

                    fire(cc + 2, fb)

                wait_gather(bb)

                @plsc.parallel_loop(0, _CH, step=1, unroll=2)
                def _(r):
                    for col in range(0, d, 16):
                        wbuf[bb, r, pl.ds(col, 16)] = (
                            wbuf[bb, r, pl.ds(col, 16)]
                            + pbuf[bb, r, pl.ds(col, 16)])

                pltpu.async_copy(
                    wbuf.at[bb], out_hbm.at[pl.ds(base + cc * _CH, _CH)],
                    semo[bb])

        for bb in range(_NB):
            wait_out(bb)

    return gather_kernel


def _ln_body(s_ref, t_ref, lw_ref, lb_ref, o_ref):
    x = s_ref[...] + t_ref[0:1, :]
    mean = jnp.mean(x, axis=-1, keepdims=True)
    m2 = jnp.mean(x * x, axis=-1, keepdims=True)
    var = m2 - mean * mean
    o_ref[...] = (x - mean) * lax.rsqrt(var + _EPS) * lw_ref[...] + lb_ref[...]


def _ln(sum_rows, type_emb, ln_w, ln_b):
    b, d = sum_rows.shape
    rb = 2048
    grid = (b // rb,)
    return pl.pallas_call(
        _ln_body,
        grid=grid,
        in_specs=[
            pl.BlockSpec((rb, d), lambda i: (i, 0)),
            pl.BlockSpec(type_emb.shape, lambda i: (0, 0)),
            pl.BlockSpec((1, d), lambda i: (0, 0)),
            pl.BlockSpec((1, d), lambda i: (0, 0)),
        ],
        out_specs=pl.BlockSpec((rb, d), lambda i: (i, 0)),
        out_shape=jax.ShapeDtypeStruct((b, d), jnp.float32),
    )(sum_rows, type_emb, ln_w, ln_b)


def kernel(input_ids, word_emb, pos_emb, type_emb, ln_w, ln_b):
    bb, seq = input_ids.shape
    d = word_emb.shape[1]
    b = bb * seq

    position_ids = _position_ids(input_ids)

    gather = _make_gather_add(word_emb.shape[0], pos_emb.shape[0], d, bb, seq)
    sum_rows = gather(word_emb, pos_emb, input_ids, position_ids)

    out = _ln(sum_rows, type_emb,
              ln_w.reshape(1, d), ln_b.reshape(1, d))
    return out.reshape(bb, seq, d)
